# Initial kernel scaffold; baseline (speedup 1.0000x reference)
import jax
import jax.numpy as jnp
from jax.experimental import pallas as pl


def kernel(x, adj_src, adj_dst, inc_u, inc_v, W1, W2, W3, W4):
    raise NotImplementedError("write your pallas kernel here")


# speedup vs baseline: 3.3512x; 3.3512x over previous
import functools

import jax
import jax.numpy as jnp
from jax import lax
from jax.experimental import pallas as pl
from jax.experimental.pallas import tpu as pltpu
from jax.experimental.pallas import tpu_sc as plsc

N = 10000
C = 128
NC = 2
NS = 16
ZR = 16
RPT = 624
TAIL = N - NS * RPT
K = 80


def _mm_pm_body(x_ref, w_ref, op_ref, on_ref):
    a = jnp.dot(x_ref[...], w_ref[...], preferred_element_type=jnp.float32)
    op_ref[...] = a
    on_ref[...] = -a


def _sig_mm_pm_body(x_ref, w_ref, op_ref, on_ref):
    s = jax.nn.sigmoid(x_ref[...])
    a = jnp.dot(s, w_ref[...], preferred_element_type=jnp.float32)
    op_ref[...] = a
    on_ref[...] = -a


def _mm_sig_body(x_ref, w_ref, o_ref):
    o_ref[...] = jax.nn.sigmoid(
        jnp.dot(x_ref[...], w_ref[...], preferred_element_type=jnp.float32))


def _mm_add_sig_body(x_ref, w_ref, b_ref, o_ref):
    o_ref[...] = jax.nn.sigmoid(
        jnp.dot(x_ref[...], w_ref[...], preferred_element_type=jnp.float32)
        + b_ref[...])


def _row_spec(blk):
    return pl.BlockSpec((blk, C), lambda i: (i, 0))


def _w_spec():
    return pl.BlockSpec((C, C), lambda i: (0, 0))


def _tc_pm(body, x, w, blk):
    rows = x.shape[0]
    return pl.pallas_call(
        body,
        grid=(rows // blk,),
        in_specs=[_row_spec(blk), _w_spec()],
        out_specs=[_row_spec(blk), _row_spec(blk)],
        out_shape=[jax.ShapeDtypeStruct((rows, C), jnp.float32)] * 2,
    )(x, w)


def _tc_mm_sig(x, w, blk):
    rows = x.shape[0]
    return pl.pallas_call(
        _mm_sig_body,
        grid=(rows // blk,),
        in_specs=[_row_spec(blk), _w_spec()],
        out_specs=_row_spec(blk),
        out_shape=jax.ShapeDtypeStruct((rows, C), jnp.float32),
    )(x, w)


def _tc_mm_add_sig(x, w, b, blk):
    rows = x.shape[0]
    return pl.pallas_call(
        _mm_add_sig_body,
        grid=(rows // blk,),
        in_specs=[_row_spec(blk), _w_spec(), _row_spec(blk)],
        out_specs=_row_spec(blk),
        out_shape=jax.ShapeDtypeStruct((rows, C), jnp.float32),
    )(x, w, b)


def _zero_acc_slice(acc, zbuf, tid):
    def zb(i, _):
        zbuf[i // (C // 16), pl.ds((i % (C // 16)) * 16, 16)] = (
            jnp.zeros((16,), jnp.float32))
        return 0

    lax.fori_loop(0, ZR * (C // 16), zb, 0)

    def zcopy(i, _):
        pltpu.sync_copy(zbuf, acc.at[pl.ds(tid * RPT + i * ZR, ZR)])
        return 0

    lax.fori_loop(0, RPT // ZR, zcopy, 0)

    @pl.when(tid == NS - 1)
    def _():
        pltpu.sync_copy(zbuf, acc.at[pl.ds(NS * RPT, TAIL)])


def _acc_writeback(acc, out_hbm, tid):
    pltpu.sync_copy(acc.at[pl.ds(tid * RPT, RPT)],
                    out_hbm.at[pl.ds(tid * RPT, RPT)])

    @pl.when(tid == NS - 1)
    def _():
        pltpu.sync_copy(acc.at[pl.ds(NS * RPT, TAIL)],
                        out_hbm.at[pl.ds(NS * RPT, TAIL)])


def _sc_stage_a(x, xw2, xw2n, adj_src, adj_dst, inc_v, inc_u):
    EA = adj_src.shape[0]
    EI = inc_v.shape[0]
    ept_a = EA // NS
    ept_i = EI // NS
    mesh = plsc.VectorSubcoreMesh(core_axis_name="c", subcore_axis_name="s")

    @functools.partial(
        pl.kernel,
        out_type=[jax.ShapeDtypeStruct((N, C), jnp.float32),
                  jax.ShapeDtypeStruct((EI, C), jnp.float32)],
        mesh=mesh,
        scratch_types=[
            pltpu.VMEM_SHARED((N, C), jnp.float32),
            pltpu.VMEM((K,), jnp.int32),
            pltpu.VMEM((K,), jnp.int32),
            pltpu.VMEM((K, C), jnp.float32),
            pltpu.VMEM((ZR, C), jnp.float32),
            pltpu.SemaphoreType.DMA,
        ],
    )
    def k(x_hbm, xw2_hbm, xw2n_hbm, asrc_hbm, adst_hbm, iv_hbm, iu_hbm,
          gx_hbm, epre_hbm, acc, sidx, didx, rows, zbuf, sem):
        cid = lax.axis_index("c")
        tid = lax.axis_index("s")

        @pl.when(cid == 0)
        def _():
            _zero_acc_slice(acc, zbuf, tid)
            plsc.subcore_barrier()

            def ch(ci, _):
                base = tid * ept_a + ci * K
                pltpu.sync_copy(asrc_hbm.at[pl.ds(base, K)], sidx)
                pltpu.sync_copy(adst_hbm.at[pl.ds(base, K)], didx)
                pltpu.async_copy(x_hbm.at[sidx], rows, sem).wait()
                pltpu.sync_copy(rows, acc.at[didx], add=True)
                return 0

            lax.fori_loop(0, ept_a // K, ch, 0)
            plsc.subcore_barrier()
            _acc_writeback(acc, gx_hbm, tid)

        @pl.when(cid == 1)
        def _():
            def ch(ci, _):
                base = tid * ept_i + ci * K
                pltpu.sync_copy(iv_hbm.at[pl.ds(base, K)], sidx)
                pltpu.sync_copy(iu_hbm.at[pl.ds(base, K)], didx)
                pltpu.async_copy(xw2_hbm.at[sidx], rows, sem).wait()
                pltpu.async_copy(xw2n_hbm.at[didx], rows, sem, add=True).wait()
                pltpu.sync_copy(rows, epre_hbm.at[pl.ds(base, K)])
                return 0

            lax.fori_loop(0, ept_i // K, ch, 0)

    return k(x, xw2, xw2n, adj_src, adj_dst, inc_v, inc_u)


def _sc_stage_c(n1, ewp, ewn, adj_src, adj_dst, inc_v, inc_u):
    EA = adj_src.shape[0]
    EI = inc_v.shape[0]
    ept_a = EA // NS
    ept_i = EI // NS
    mesh = plsc.VectorSubcoreMesh(core_axis_name="c", subcore_axis_name="s")

    @functools.partial(
        pl.kernel,
        out_type=[jax.ShapeDtypeStruct((N, C), jnp.float32),
                  jax.ShapeDtypeStruct((N, C), jnp.float32)],
        mesh=mesh,
        scratch_types=[
            pltpu.VMEM_SHARED((N, C), jnp.float32),
            pltpu.VMEM((K,), jnp.int32),
            pltpu.VMEM((K,), jnp.int32),
            pltpu.VMEM((K, C), jnp.float32),
            pltpu.VMEM((ZR, C), jnp.float32),
            pltpu.SemaphoreType.DMA,
        ],
    )
    def k(n1_hbm, ewp_hbm, ewn_hbm, asrc_hbm, adst_hbm, iv_hbm, iu_hbm,
          gn_hbm, e2_hbm, acc, sidx, didx, rows, zbuf, sem):
        cid = lax.axis_index("c")
        tid = lax.axis_index("s")

        _zero_acc_slice(acc, zbuf, tid)
        plsc.subcore_barrier()

        @pl.when(cid == 0)
        def _():
            def ch(ci, _):
                base = tid * ept_a + ci * K
                pltpu.sync_copy(asrc_hbm.at[pl.ds(base, K)], sidx)
                pltpu.sync_copy(adst_hbm.at[pl.ds(base, K)], didx)
                pltpu.async_copy(n1_hbm.at[sidx], rows, sem).wait()
                pltpu.sync_copy(rows, acc.at[didx], add=True)
                return 0

            lax.fori_loop(0, ept_a // K, ch, 0)
            plsc.subcore_barrier()
            _acc_writeback(acc, gn_hbm, tid)

        @pl.when(cid == 1)
        def _():
            def ch(ci, _):
                base = tid * ept_i + ci * K
                pltpu.sync_copy(iv_hbm.at[pl.ds(base, K)], sidx)
                pltpu.sync_copy(iu_hbm.at[pl.ds(base, K)], didx)
                pltpu.sync_copy(ewp_hbm.at[pl.ds(base, K)], rows)
                pltpu.sync_copy(rows, acc.at[sidx], add=True)
                pltpu.sync_copy(ewn_hbm.at[pl.ds(base, K)], rows)
                pltpu.sync_copy(rows, acc.at[didx], add=True)
                return 0

            lax.fori_loop(0, ept_i // K, ch, 0)
            plsc.subcore_barrier()
            _acc_writeback(acc, e2_hbm, tid)

    return k(n1, ewp, ewn, adj_src, adj_dst, inc_v, inc_u)


def kernel(x, adj_src, adj_dst, inc_u, inc_v, W1, W2, W3, W4):
    xw2, xw2n = _tc_pm(_mm_pm_body, x, W2, blk=1000)
    gx, e_pre = _sc_stage_a(x, xw2, xw2n, adj_src, adj_dst, inc_v, inc_u)
    n1 = _tc_mm_sig(gx, W1, blk=1000)
    ewp, ewn = _tc_pm(_sig_mm_pm_body, e_pre, W4, blk=2000)
    gn, e2 = _sc_stage_c(n1, ewp, ewn, adj_src, adj_dst, inc_v, inc_u)
    return _tc_mm_add_sig(gn, W3, e2, blk=1000)


# speedup vs baseline: 5.9771x; 1.7836x over previous
import functools

import jax
import jax.numpy as jnp
from jax import lax
from jax.experimental import pallas as pl
from jax.experimental.pallas import tpu as pltpu
from jax.experimental.pallas import tpu_sc as plsc

N = 10000
C = 128
NC = 2
NS = 16
ZR = 48
RPT = 624
TAIL = N - NS * RPT
K = 80


def _mm_pm_body(x_ref, w_ref, op_ref, on_ref):
    a = jnp.dot(x_ref[...], w_ref[...], preferred_element_type=jnp.float32)
    op_ref[...] = a
    on_ref[...] = -a


def _sig_mm_pm_body(x_ref, w_ref, op_ref, on_ref):
    s = jax.nn.sigmoid(x_ref[...])
    a = jnp.dot(s, w_ref[...], preferred_element_type=jnp.float32)
    op_ref[...] = a
    on_ref[...] = -a


def _mm_sig_body(x_ref, w_ref, o_ref):
    o_ref[...] = jax.nn.sigmoid(
        jnp.dot(x_ref[...], w_ref[...], preferred_element_type=jnp.float32))


def _mm_add_sig_body(x_ref, w_ref, b_ref, o_ref):
    o_ref[...] = jax.nn.sigmoid(
        jnp.dot(x_ref[...], w_ref[...], preferred_element_type=jnp.float32)
        + b_ref[...])


def _row_spec(blk):
    return pl.BlockSpec((blk, C), lambda i: (i, 0))


def _w_spec():
    return pl.BlockSpec((C, C), lambda i: (0, 0))


def _tc_pm(body, x, w, blk):
    rows = x.shape[0]
    return pl.pallas_call(
        body,
        grid=(rows // blk,),
        in_specs=[_row_spec(blk), _w_spec()],
        out_specs=[_row_spec(blk), _row_spec(blk)],
        out_shape=[jax.ShapeDtypeStruct((rows, C), jnp.float32)] * 2,
    )(x, w)


def _tc_mm_sig(x, w, blk):
    rows = x.shape[0]
    return pl.pallas_call(
        _mm_sig_body,
        grid=(rows // blk,),
        in_specs=[_row_spec(blk), _w_spec()],
        out_specs=_row_spec(blk),
        out_shape=jax.ShapeDtypeStruct((rows, C), jnp.float32),
    )(x, w)


def _tc_mm_add_sig(x, w, b, blk):
    rows = x.shape[0]
    return pl.pallas_call(
        _mm_add_sig_body,
        grid=(rows // blk,),
        in_specs=[_row_spec(blk), _w_spec(), _row_spec(blk)],
        out_specs=_row_spec(blk),
        out_shape=jax.ShapeDtypeStruct((rows, C), jnp.float32),
    )(x, w, b)


def _zero_acc_slice(acc, zbuf, tid, sem):
    def zb(i, _):
        zbuf[i // (C // 16), pl.ds((i % (C // 16)) * 16, 16)] = (
            jnp.zeros((16,), jnp.float32))
        return 0

    lax.fori_loop(0, ZR * (C // 16), zb, 0)

    def zcopy(i, _):
        pltpu.async_copy(zbuf, acc.at[pl.ds(tid * RPT + i * ZR, ZR)], sem)
        return 0

    lax.fori_loop(0, RPT // ZR, zcopy, 0)

    def zdrain(i, _):
        pltpu.make_async_copy(zbuf, acc.at[pl.ds(tid * RPT + i * ZR, ZR)],
                              sem).wait()
        return 0

    lax.fori_loop(0, RPT // ZR, zdrain, 0)

    @pl.when(tid == NS - 1)
    def _():
        pltpu.sync_copy(zbuf.at[pl.ds(0, TAIL)], acc.at[pl.ds(NS * RPT, TAIL)])


def _acc_writeback(acc, out_hbm, tid):
    pltpu.sync_copy(acc.at[pl.ds(tid * RPT, RPT)],
                    out_hbm.at[pl.ds(tid * RPT, RPT)])

    @pl.when(tid == NS - 1)
    def _():
        pltpu.sync_copy(acc.at[pl.ds(NS * RPT, TAIL)],
                        out_hbm.at[pl.ds(NS * RPT, TAIL)])


def _seg_sum_core(table_hbm, asrc_hbm, adst_hbm, out_hbm, acc, iall, didx0,
                  didx1, rows0, rows1, zbuf, sem0, sem1, semz, tid, ept):
    nch = ept // K
    base_t = tid * ept
    _zero_acc_slice(acc, zbuf, tid, semz)
    pltpu.sync_copy(asrc_hbm.at[pl.ds(base_t, ept)], iall.at[pl.ds(0, ept)])
    plsc.subcore_barrier()

    pltpu.sync_copy(adst_hbm.at[pl.ds(base_t, K)], didx0)
    pltpu.async_copy(table_hbm.at[iall.at[pl.ds(0, K)]], rows0, sem0)

    def body(g, _):
        i0 = 2 * g
        i1 = i0 + 1
        pltpu.make_async_copy(
            table_hbm.at[iall.at[pl.ds(i0 * K, K)]], rows0, sem0).wait()
        pltpu.async_copy(table_hbm.at[iall.at[pl.ds(i1 * K, K)]], rows1, sem1)
        pltpu.sync_copy(adst_hbm.at[pl.ds(base_t + i1 * K, K)], didx1)
        pltpu.sync_copy(rows0, acc.at[didx0], add=True)
        pltpu.make_async_copy(
            table_hbm.at[iall.at[pl.ds(i1 * K, K)]], rows1, sem1).wait()

        @pl.when(i1 + 1 < nch)
        def _():
            pltpu.async_copy(
                table_hbm.at[iall.at[pl.ds((i1 + 1) * K, K)]], rows0, sem0)
            pltpu.sync_copy(adst_hbm.at[pl.ds(base_t + (i1 + 1) * K, K)],
                            didx0)

        pltpu.sync_copy(rows1, acc.at[didx1], add=True)
        return 0

    lax.fori_loop(0, nch // 2, body, 0)
    plsc.subcore_barrier()
    _acc_writeback(acc, out_hbm, tid)


def _gather_diff_core(pos_hbm, neg_hbm, iv_hbm, iu_hbm, out_hbm, iall, rows0,
                      rows1, sem0, sem1, semw, tid, ept):
    nch = ept // K
    base_t = tid * ept
    pltpu.sync_copy(iv_hbm.at[pl.ds(base_t, ept)], iall.at[pl.ds(0, ept)])
    pltpu.sync_copy(iu_hbm.at[pl.ds(base_t, ept)], iall.at[pl.ds(ept, ept)])

    def chunk(i, rows, sem):
        pltpu.async_copy(pos_hbm.at[iall.at[pl.ds(i * K, K)]], rows,
                         sem).wait()
        pltpu.async_copy(neg_hbm.at[iall.at[pl.ds(ept + i * K, K)]], rows,
                         sem, add=True).wait()
        pltpu.async_copy(rows, out_hbm.at[pl.ds(base_t + i * K, K)], semw)

    def body(g, _):
        i0 = 2 * g

        @pl.when(g > 0)
        def _():
            pltpu.make_async_copy(
                rows0, out_hbm.at[pl.ds(base_t, K)], semw).wait()
            pltpu.make_async_copy(
                rows1, out_hbm.at[pl.ds(base_t, K)], semw).wait()

        chunk(i0, rows0, sem0)
        chunk(i0 + 1, rows1, sem1)
        return 0

    lax.fori_loop(0, nch // 2, body, 0)
    pltpu.make_async_copy(rows0, out_hbm.at[pl.ds(base_t, K)], semw).wait()
    pltpu.make_async_copy(rows1, out_hbm.at[pl.ds(base_t, K)], semw).wait()

    @pl.when((nch % 2) == 1)
    def _():
        i = nch - 1
        pltpu.async_copy(pos_hbm.at[iall.at[pl.ds(i * K, K)]], rows0,
                         sem0).wait()
        pltpu.async_copy(neg_hbm.at[iall.at[pl.ds(ept + i * K, K)]], rows0,
                         sem0, add=True).wait()
        pltpu.sync_copy(rows0, out_hbm.at[pl.ds(base_t + i * K, K)])


def _scatter_pm_core(ewp_hbm, ewn_hbm, iv_hbm, iu_hbm, out_hbm, acc, vidx,
                     uidx, rows0, rows1, zbuf, sem0, sem1, semz, tid, ept):
    nch = ept // K
    base_t = tid * ept
    _zero_acc_slice(acc, zbuf, tid, semz)
    plsc.subcore_barrier()

    pltpu.async_copy(ewp_hbm.at[pl.ds(base_t, K)], rows0, sem0)
    pltpu.sync_copy(iv_hbm.at[pl.ds(base_t, K)], vidx)

    def body(i, _):
        b = base_t + i * K
        bn = b + K
        pltpu.make_async_copy(ewp_hbm.at[pl.ds(b, K)], rows0, sem0).wait()
        pltpu.async_copy(ewn_hbm.at[pl.ds(b, K)], rows1, sem1)
        pltpu.sync_copy(iu_hbm.at[pl.ds(b, K)], uidx)
        pltpu.sync_copy(rows0, acc.at[vidx], add=True)
        pltpu.make_async_copy(ewn_hbm.at[pl.ds(b, K)], rows1, sem1).wait()

        @pl.when(i + 1 < nch)
        def _():
            pltpu.async_copy(ewp_hbm.at[pl.ds(bn, K)], rows0, sem0)
            pltpu.sync_copy(iv_hbm.at[pl.ds(bn, K)], vidx)

        pltpu.sync_copy(rows1, acc.at[uidx], add=True)
        return 0

    lax.fori_loop(0, nch, body, 0)
    plsc.subcore_barrier()
    _acc_writeback(acc, out_hbm, tid)


def _sc_stage_a(x, xw2, xw2n, adj_src, adj_dst, inc_v, inc_u):
    EA = adj_src.shape[0]
    EI = inc_v.shape[0]
    ept_a = EA // NS
    ept_i = EI // NS
    mesh = plsc.VectorSubcoreMesh(core_axis_name="c", subcore_axis_name="s")

    @functools.partial(
        pl.kernel,
        out_type=[jax.ShapeDtypeStruct((N, C), jnp.float32),
                  jax.ShapeDtypeStruct((EI, C), jnp.float32)],
        mesh=mesh,
        scratch_types=[
            pltpu.VMEM_SHARED((N, C), jnp.float32),
            pltpu.VMEM((2 * ept_i,), jnp.int32),
            pltpu.VMEM((K,), jnp.int32),
            pltpu.VMEM((K,), jnp.int32),
            pltpu.VMEM((K, C), jnp.float32),
            pltpu.VMEM((K, C), jnp.float32),
            pltpu.VMEM((ZR, C), jnp.float32),
            pltpu.SemaphoreType.DMA,
            pltpu.SemaphoreType.DMA,
            pltpu.SemaphoreType.DMA,
            pltpu.SemaphoreType.DMA,
        ],
    )
    def k(x_hbm, xw2_hbm, xw2n_hbm, asrc_hbm, adst_hbm, iv_hbm, iu_hbm,
          gx_hbm, epre_hbm, acc, iall, didx0, didx1, rows0, rows1, zbuf,
          sem0, sem1, semz, semw):
        cid = lax.axis_index("c")
        tid = lax.axis_index("s")

        @pl.when(cid == 0)
        def _():
            _seg_sum_core(x_hbm, asrc_hbm, adst_hbm, gx_hbm, acc, iall,
                          didx0, didx1, rows0, rows1, zbuf, sem0, sem1,
                          semz, tid, ept_a)

        @pl.when(cid == 1)
        def _():
            _gather_diff_core(xw2_hbm, xw2n_hbm, iv_hbm, iu_hbm, epre_hbm,
                              iall, rows0, rows1, sem0, sem1, semw, tid,
                              ept_i)

    return k(x, xw2, xw2n, adj_src, adj_dst, inc_v, inc_u)


def _sc_stage_c(n1, ewp, ewn, adj_src, adj_dst, inc_v, inc_u):
    EA = adj_src.shape[0]
    EI = inc_v.shape[0]
    ept_a = EA // NS
    ept_i = EI // NS
    mesh = plsc.VectorSubcoreMesh(core_axis_name="c", subcore_axis_name="s")

    @functools.partial(
        pl.kernel,
        out_type=[jax.ShapeDtypeStruct((N, C), jnp.float32),
                  jax.ShapeDtypeStruct((N, C), jnp.float32)],
        mesh=mesh,
        scratch_types=[
            pltpu.VMEM_SHARED((N, C), jnp.float32),
            pltpu.VMEM((ept_a,), jnp.int32),
            pltpu.VMEM((K,), jnp.int32),
            pltpu.VMEM((K,), jnp.int32),
            pltpu.VMEM((K, C), jnp.float32),
            pltpu.VMEM((K, C), jnp.float32),
            pltpu.VMEM((ZR, C), jnp.float32),
            pltpu.SemaphoreType.DMA,
            pltpu.SemaphoreType.DMA,
            pltpu.SemaphoreType.DMA,
        ],
    )
    def k(n1_hbm, ewp_hbm, ewn_hbm, asrc_hbm, adst_hbm, iv_hbm, iu_hbm,
          gn_hbm, e2_hbm, acc, iall, didx0, didx1, rows0, rows1, zbuf,
          sem0, sem1, semz):
        cid = lax.axis_index("c")
        tid = lax.axis_index("s")

        @pl.when(cid == 0)
        def _():
            _seg_sum_core(n1_hbm, asrc_hbm, adst_hbm, gn_hbm, acc, iall,
                          didx0, didx1, rows0, rows1, zbuf, sem0, sem1,
                          semz, tid, ept_a)

        @pl.when(cid == 1)
        def _():
            _scatter_pm_core(ewp_hbm, ewn_hbm, iv_hbm, iu_hbm, e2_hbm, acc,
                             didx0, didx1, rows0, rows1, zbuf, sem0, sem1,
                             semz, tid, ept_i)

    return k(n1, ewp, ewn, adj_src, adj_dst, inc_v, inc_u)


def kernel(x, adj_src, adj_dst, inc_u, inc_v, W1, W2, W3, W4):
    xw2, xw2n = _tc_pm(_mm_pm_body, x, W2, blk=1000)
    gx, e_pre = _sc_stage_a(x, xw2, xw2n, adj_src, adj_dst, inc_v, inc_u)
    n1 = _tc_mm_sig(gx, W1, blk=1000)
    ewp, ewn = _tc_pm(_sig_mm_pm_body, e_pre, W4, blk=2000)
    gn, e2 = _sc_stage_c(n1, ewp, ewn, adj_src, adj_dst, inc_v, inc_u)
    return _tc_mm_add_sig(gn, W3, e2, blk=1000)
